# baseline (speedup 1.0000x reference)
import functools

import jax
import jax.numpy as jnp
from jax import lax
from jax.experimental import pallas as pl
from jax.experimental.pallas import tpu as pltpu
from jax.experimental.pallas import tpu_sc as plsc

B = 16384
H = 4
D = 64
NC = 2
NS = 16
NW = NC * NS
S_PER_W = B // NW
C = 128
G = S_PER_W // C


def _sc_body(ids_hbm, table_hbm, out_hbm, idx_v, rows_v, pooled_v, sem):
    wid = lax.axis_index("s") * NC + lax.axis_index("c")

    for g in range(G):
        base = wid * S_PER_W + g * C
        for h in range(H):
            pltpu.sync_copy(ids_hbm.at[pl.ds(h * B + base, C)], idx_v.at[h])
        copies = [
            pltpu.make_async_copy(table_hbm.at[idx_v.at[h]], rows_v.at[h], sem)
            for h in range(H)
        ]
        for cp in copies:
            cp.start()
        for cp in copies:
            cp.wait()

        def pool_row(s, _):
            for d in range(D // 16):
                sl = pl.ds(d * 16, 16)
                acc = rows_v[0, s, sl]
                acc = acc + rows_v[1, s, sl]
                acc = acc + rows_v[2, s, sl]
                acc = acc + rows_v[3, s, sl]
                pooled_v[s, sl] = acc
            return _

        lax.fori_loop(0, C, pool_row, 0, unroll=2)

        pltpu.sync_copy(pooled_v, out_hbm.at[pl.ds(base, C)])


_sc_gather_pool = functools.partial(
    pl.kernel,
    out_type=jax.ShapeDtypeStruct((B, D), jnp.float32),
    mesh=plsc.VectorSubcoreMesh(core_axis_name="c", subcore_axis_name="s"),
    scratch_types=[
        pltpu.VMEM((H, C), jnp.int32),
        pltpu.VMEM((H, C, D), jnp.float32),
        pltpu.VMEM((C, D), jnp.float32),
        pltpu.SemaphoreType.DMA,
    ],
    compiler_params=pltpu.CompilerParams(use_tc_tiling_on_sc=False),
)(_sc_body)


def _tc_body(x_ref, w_ref, b_ref, o_ref):
    y = jnp.dot(x_ref[...], w_ref[...], preferred_element_type=jnp.float32)
    o_ref[...] = jnp.maximum(y + b_ref[...], 0.0)


def _tc_linear(x, w, b):
    blk = 2048
    return pl.pallas_call(
        _tc_body,
        grid=(B // blk,),
        in_specs=[
            pl.BlockSpec((blk, D), lambda i: (i, 0)),
            pl.BlockSpec((D, D), lambda i: (0, 0)),
            pl.BlockSpec((1, D), lambda i: (0, 0)),
        ],
        out_specs=pl.BlockSpec((blk, D), lambda i: (i, 0)),
        out_shape=jax.ShapeDtypeStruct((B, D), jnp.float32),
    )(x, w, b)


def kernel(ids, emb_table, proj_w, proj_b):
    ids_t = ids.T.reshape(-1)
    pooled = _sc_gather_pool(ids_t, emb_table)
    wt = proj_w.T * (1.0 / H)
    return _tc_linear(pooled, wt, proj_b.reshape(1, D))
